# baseline (speedup 1.0000x reference)
import jax
import jax.numpy as jnp
from jax import lax
from jax.experimental import pallas as pl
from jax.experimental.pallas import tpu as pltpu
from jax.experimental.pallas import tpu_sc as plsc

N = 10000
E = 320000
F = 128
NC = 2
NS = 16
NW = NC * NS
C = 80
CH = 128
EPAD = NW * CH * C
NPAD = 10240
RPT = NPAD // NS
RCH = RPT // C
EPS = 1e-5

def _deg_body(dst_hbm, out_hbm, dst_v, ones_v, zer_v, acc_sh):
    c = lax.axis_index("c")
    s = lax.axis_index("s")
    w = c * NS + s
    for q in range(C // 16):
        ones_v[pl.ds(q * 16, 16)] = jnp.ones((16,), jnp.float32)

    def _zb(i, carry):
        zer_v[pl.ds(i * 16, 16)] = jnp.zeros((16,), jnp.float32)
        return carry

    lax.fori_loop(0, RPT // 16, _zb, 0)
    pltpu.sync_copy(zer_v, acc_sh.at[pl.ds(s * RPT, RPT)])
    plsc.subcore_barrier()
    pltpu.sync_copy(dst_hbm.at[w], dst_v)

    def _sb(j, carry):
        pltpu.sync_copy(ones_v, acc_sh.at[dst_v.at[j]], add=True)
        return carry

    lax.fori_loop(0, CH, _sb, 0)
    plsc.subcore_barrier()
    pltpu.sync_copy(acc_sh.at[pl.ds(s * RPT, RPT)], out_hbm.at[c, pl.ds(s * RPT, RPT)])


def _scatter_body(g_hbm, eidx_hbm, out_hbm, idx_v, rows_v, acc_sh, gsem0, gsem1, isem0, isem1):
    c = lax.axis_index("c")
    s = lax.axis_index("s")
    w = c * NS + s
    gsems = (gsem0, gsem1)
    isems = (isem0, isem1)

    def _zb(i, carry):
        for q in range(F // 16):
            rows_v[0, i, pl.ds(q * 16, 16)] = jnp.zeros((16,), jnp.float32)
        return carry

    lax.fori_loop(0, C, _zb, 0)
    for k in range(RCH):
        pltpu.sync_copy(rows_v.at[0], acc_sh.at[pl.ds(s * RPT + k * C, C)])
    plsc.subcore_barrier()

    def _idx_start(j, b):
        pltpu.async_copy(eidx_hbm.at[w, j], idx_v.at[b], isems[b])

    def _idx_wait(j, b):
        pltpu.make_async_copy(eidx_hbm.at[w, j], idx_v.at[b], isems[b]).wait()

    def _gather_start(b):
        pltpu.async_copy(g_hbm.at[idx_v.at[b, 0]], rows_v.at[b], gsems[b])

    def _gather_wait(b):
        pltpu.make_async_copy(g_hbm.at[idx_v.at[b, 0]], rows_v.at[b], gsems[b]).wait()

    def _scatter(b):
        pltpu.sync_copy(rows_v.at[b], acc_sh.at[idx_v.at[b, 1]], add=True)

    _idx_start(0, 0)
    _idx_start(1, 1)
    _idx_wait(0, 0)
    _gather_start(0)

    def _mb(i, carry):
        j2 = i * 2
        for b in range(2):
            j = j2 + b
            nb = 1 - b
            _idx_wait(j + 1, nb)
            _gather_start(nb)
            _gather_wait(b)
            _scatter(b)
            _idx_start(j + 2, b)
        return carry

    lax.fori_loop(0, CH // 2 - 1, _mb, 0)
    _idx_wait(CH - 1, 1)
    _gather_start(1)
    _gather_wait(0)
    _scatter(0)
    _gather_wait(1)
    _scatter(1)
    plsc.subcore_barrier()
    for k in range(RCH):
        pltpu.sync_copy(
            acc_sh.at[pl.ds(s * RPT + k * C, C)],
            out_hbm.at[c, pl.ds(s * RPT + k * C, C)],
        )


_sc_calls_cache = {}


def _sc_calls():
    if "deg" not in _sc_calls_cache:
        mesh = plsc.VectorSubcoreMesh(
            core_axis_name="c", subcore_axis_name="s", num_cores=NC, num_subcores=NS
        )
        _sc_calls_cache["deg"] = pl.kernel(
            _deg_body,
            out_type=jax.ShapeDtypeStruct((NC, NPAD), jnp.float32),
            mesh=mesh,
            scratch_types=[
                pltpu.VMEM((CH, C), jnp.int32),
                pltpu.VMEM((C,), jnp.float32),
                pltpu.VMEM((RPT,), jnp.float32),
                pltpu.VMEM_SHARED((NPAD,), jnp.float32),
            ],
        )
        _sc_calls_cache["scatter"] = pl.kernel(
            _scatter_body,
            out_type=jax.ShapeDtypeStruct((NC, NPAD, F), jnp.float32),
            mesh=mesh,
            scratch_types=[
                pltpu.VMEM((2, 2, C), jnp.int32),
                pltpu.VMEM((2, C, F), jnp.float32),
                pltpu.VMEM_SHARED((NPAD, F), jnp.float32),
                pltpu.SemaphoreType.DMA,
                pltpu.SemaphoreType.DMA,
                pltpu.SemaphoreType.DMA,
                pltpu.SemaphoreType.DMA,
            ],
        )
    return _sc_calls_cache["deg"], _sc_calls_cache["scatter"]


def _dinv_from(degp_ref):
    deg = degp_ref[0, :N] + degp_ref[1, :N] + 1.0
    return lax.rsqrt(deg)


def _tc1_body(x_ref, w1_ref, degp_ref, g1_ref):
    dinv = _dinv_from(degp_ref)
    h = jnp.dot(x_ref[...], w1_ref[...], preferred_element_type=jnp.float32)
    g1_ref[...] = h * dinv[:, None]


_tc1 = pl.pallas_call(
    _tc1_body,
    out_shape=jax.ShapeDtypeStruct((N, F), jnp.float32),
)


def _tc2_body(accp_ref, g1_ref, degp_ref, b1_ref, gamma_ref, beta_ref, w2_ref, g2_ref):
    dinv = _dinv_from(degp_ref)
    acc = accp_ref[0, :N, :] + accp_ref[1, :N, :] + g1_ref[...]
    t = acc * dinv[:, None] + b1_ref[...]
    mean = jnp.mean(t, axis=0)
    var = jnp.mean((t - mean) ** 2, axis=0)
    t = (t - mean) * lax.rsqrt(var + EPS) * gamma_ref[...] + beta_ref[...]
    t = jnp.maximum(t, 0.0)
    h2 = jnp.dot(t, w2_ref[...], preferred_element_type=jnp.float32)
    g2_ref[...] = h2 * dinv[:, None]


_tc2 = pl.pallas_call(
    _tc2_body,
    out_shape=jax.ShapeDtypeStruct((N, F), jnp.float32),
)


def _tc3_body(accp_ref, g2_ref, degp_ref, b2_ref, out_ref):
    dinv = _dinv_from(degp_ref)
    acc = accp_ref[0, :N, :] + accp_ref[1, :N, :] + g2_ref[...]
    out_ref[...] = acc * dinv[:, None] + b2_ref[...]


_tc3 = pl.pallas_call(
    _tc3_body,
    out_shape=jax.ShapeDtypeStruct((N, F), jnp.float32),
)


def kernel(x, edge_index, W1, b1, W2, b2, gamma, beta):
    src = edge_index[0].astype(jnp.int32)
    dst = edge_index[1].astype(jnp.int32)
    pad = EPAD - E
    pad_i = jnp.arange(pad, dtype=jnp.int32)
    src_p = jnp.concatenate([src, pad_i % N]).reshape(NW, CH, C)
    dst_p = jnp.concatenate([dst, N + pad_i % (NPAD - N)]).reshape(NW, CH, C)
    eidx = jnp.stack([src_p, dst_p], axis=2)

    deg_call, scatter_call = _sc_calls()
    degp = deg_call(dst_p)
    g1 = _tc1(x, W1, degp)
    acc1 = scatter_call(g1, eidx)
    g2 = _tc2(acc1, g1, degp, b1, gamma, beta, W2)
    acc2 = scatter_call(g2, eidx)
    return _tc3(acc2, g2, degp, b2)


# speedup vs baseline: 1.3433x; 1.3433x over previous
import jax
import jax.numpy as jnp
from jax import lax
from jax.experimental import pallas as pl
from jax.experimental.pallas import tpu as pltpu
from jax.experimental.pallas import tpu_sc as plsc

N = 10000
E = 320000
F = 128
NC = 2
NS = 16
NW = NC * NS
C = 112
CH = 93
CHP = CH + 2
EPAD = NW * CH * C
NPAD = 10240
RPT = NPAD // NS
RCHUNKS = [(o, min(C, RPT - o)) for o in range(0, RPT, C)]
EPS = 1e-5

def _deg_body(dst_hbm, out_hbm, dst_v, ones_v, zer_v, acc_sh):
    c = lax.axis_index("c")
    s = lax.axis_index("s")
    w = c * NS + s
    for q in range(C // 16):
        ones_v[pl.ds(q * 16, 16)] = jnp.ones((16,), jnp.float32)

    def _zb(i, carry):
        zer_v[pl.ds(i * 16, 16)] = jnp.zeros((16,), jnp.float32)
        return carry

    lax.fori_loop(0, RPT // 16, _zb, 0)
    pltpu.sync_copy(zer_v, acc_sh.at[pl.ds(s * RPT, RPT)])
    plsc.subcore_barrier()
    pltpu.sync_copy(dst_hbm.at[w], dst_v)

    def _sb(j, carry):
        pltpu.sync_copy(ones_v, acc_sh.at[dst_v.at[j]], add=True)
        return carry

    lax.fori_loop(0, CH, _sb, 0)
    plsc.subcore_barrier()
    pltpu.sync_copy(acc_sh.at[pl.ds(s * RPT, RPT)], out_hbm.at[c, pl.ds(s * RPT, RPT)])


def _scatter_body(g_hbm, eidx_hbm, out_hbm, idx_v, rows_v, acc_sh,
                  gsem0, gsem1, gsem2, ssem0, ssem1, ssem2,
                  isem0, isem1, isem2, isem3, isem4):
    c = lax.axis_index("c")
    s = lax.axis_index("s")
    w = c * NS + s
    gsems = (gsem0, gsem1, gsem2)
    ssems = (ssem0, ssem1, ssem2)
    isems = (isem0, isem1, isem2, isem3, isem4)

    def _zb(i, carry):
        for q in range(F // 16):
            rows_v[0, i, pl.ds(q * 16, 16)] = jnp.zeros((16,), jnp.float32)
        return carry

    lax.fori_loop(0, C, _zb, 0)
    for off, size in RCHUNKS:
        pltpu.sync_copy(
            rows_v.at[0, pl.ds(0, size)], acc_sh.at[pl.ds(s * RPT + off, size)]
        )
    plsc.subcore_barrier()

    def _idx_start(j, bi):
        pltpu.async_copy(eidx_hbm.at[w, j], idx_v.at[bi], isems[bi])

    def _idx_wait(j, bi):
        pltpu.make_async_copy(eidx_hbm.at[w, j], idx_v.at[bi], isems[bi]).wait()

    def _gather_start(br, bi):
        pltpu.async_copy(g_hbm.at[idx_v.at[bi, 0]], rows_v.at[br], gsems[br])

    def _gather_wait(br, bi):
        pltpu.make_async_copy(g_hbm.at[idx_v.at[bi, 0]], rows_v.at[br], gsems[br]).wait()

    def _scatter_start(br, bi):
        pltpu.async_copy(rows_v.at[br], acc_sh.at[idx_v.at[bi, 1]], ssems[br], add=True)

    def _scatter_wait(br, bi):
        pltpu.make_async_copy(rows_v.at[br], acc_sh.at[idx_v.at[bi, 1]], ssems[br]).wait()

    _idx_start(0, 0)
    _idx_start(1, 1)
    _idx_wait(0, 0)
    _gather_start(0, 0)
    _idx_start(2, 2)
    for m in (1, 2):
        _idx_wait(m, m)
        _gather_start(m % 3, m)
        _gather_wait((m - 1) % 3, m - 1)
        _scatter_start((m - 1) % 3, m - 1)
        _idx_start(m + 2, m + 2)

    def _step(m, q):
        rb, ib = q % 3, q % 5
        pb, pi = (q - 1) % 3, (q - 1) % 5
        ni = (q + 2) % 5
        _scatter_wait(rb, ni)
        _idx_wait(m, ib)
        _gather_start(rb, ib)
        _gather_wait(pb, pi)
        _scatter_start(pb, pi)
        _idx_start(m + 2, ni)

    def _mb(i, carry):
        base = 3 + 15 * i
        for qq in range(15):
            _step(base + qq, (3 + qq) % 15)
        return carry

    lax.fori_loop(0, (CH - 3) // 15, _mb, 0)
    _gather_wait((CH - 1) % 3, (CH - 1) % 5)
    _scatter_start((CH - 1) % 3, (CH - 1) % 5)
    for m in (CH - 3, CH - 2, CH - 1):
        _scatter_wait(m % 3, m % 5)
    plsc.subcore_barrier()
    for off, size in RCHUNKS:
        pltpu.sync_copy(
            acc_sh.at[pl.ds(s * RPT + off, size)],
            out_hbm.at[c, pl.ds(s * RPT + off, size)],
        )


_sc_calls_cache = {}


def _sc_calls():
    if "deg" not in _sc_calls_cache:
        mesh = plsc.VectorSubcoreMesh(
            core_axis_name="c", subcore_axis_name="s", num_cores=NC, num_subcores=NS
        )
        _sc_calls_cache["deg"] = pl.kernel(
            _deg_body,
            out_type=jax.ShapeDtypeStruct((NC, NPAD), jnp.float32),
            mesh=mesh,
            scratch_types=[
                pltpu.VMEM((CH, C), jnp.int32),
                pltpu.VMEM((C,), jnp.float32),
                pltpu.VMEM((RPT,), jnp.float32),
                pltpu.VMEM_SHARED((NPAD,), jnp.float32),
            ],
        )
        _sc_calls_cache["scatter"] = pl.kernel(
            _scatter_body,
            out_type=jax.ShapeDtypeStruct((NC, NPAD, F), jnp.float32),
            mesh=mesh,
            scratch_types=[
                pltpu.VMEM((5, 2, C), jnp.int32),
                pltpu.VMEM((3, C, F), jnp.float32),
                pltpu.VMEM_SHARED((NPAD, F), jnp.float32),
            ] + [pltpu.SemaphoreType.DMA] * 11,
        )
    return _sc_calls_cache["deg"], _sc_calls_cache["scatter"]


def _dinv_from(degp_ref):
    deg = degp_ref[0, :N] + degp_ref[1, :N] + 1.0
    return lax.rsqrt(deg)


def _tc1_body(x_ref, w1_ref, degp_ref, g1_ref):
    dinv = _dinv_from(degp_ref)
    h = jnp.dot(x_ref[...], w1_ref[...], preferred_element_type=jnp.float32)
    g1_ref[...] = h * dinv[:, None]


_tc1 = pl.pallas_call(
    _tc1_body,
    out_shape=jax.ShapeDtypeStruct((N, F), jnp.float32),
)


def _tc2_body(accp_ref, g1_ref, degp_ref, b1_ref, gamma_ref, beta_ref, w2_ref, g2_ref):
    dinv = _dinv_from(degp_ref)
    acc = accp_ref[0, :N, :] + accp_ref[1, :N, :] + g1_ref[...]
    t = acc * dinv[:, None] + b1_ref[...]
    mean = jnp.mean(t, axis=0)
    var = jnp.mean((t - mean) ** 2, axis=0)
    t = (t - mean) * lax.rsqrt(var + EPS) * gamma_ref[...] + beta_ref[...]
    t = jnp.maximum(t, 0.0)
    h2 = jnp.dot(t, w2_ref[...], preferred_element_type=jnp.float32)
    g2_ref[...] = h2 * dinv[:, None]


_tc2 = pl.pallas_call(
    _tc2_body,
    out_shape=jax.ShapeDtypeStruct((N, F), jnp.float32),
)


def _tc3_body(accp_ref, g2_ref, degp_ref, b2_ref, out_ref):
    dinv = _dinv_from(degp_ref)
    acc = accp_ref[0, :N, :] + accp_ref[1, :N, :] + g2_ref[...]
    out_ref[...] = acc * dinv[:, None] + b2_ref[...]


_tc3 = pl.pallas_call(
    _tc3_body,
    out_shape=jax.ShapeDtypeStruct((N, F), jnp.float32),
)


def kernel(x, edge_index, W1, b1, W2, b2, gamma, beta):
    src = edge_index[0].astype(jnp.int32)
    dst = edge_index[1].astype(jnp.int32)
    pad = EPAD - E
    pad_i = jnp.arange(pad, dtype=jnp.int32)
    src_p = jnp.concatenate([src, pad_i % N]).reshape(NW, CH, C)
    dst_p = jnp.concatenate([dst, N + pad_i % (NPAD - N)]).reshape(NW, CH, C)
    eidx = jnp.stack([src_p, dst_p], axis=2)
    eidx = jnp.concatenate(
        [eidx, jnp.zeros((NW, CHP - CH, 2, C), jnp.int32)], axis=1
    )

    deg_call, scatter_call = _sc_calls()
    degp = deg_call(dst_p)
    g1 = _tc1(x, W1, degp)
    acc1 = scatter_call(g1, eidx)
    g2 = _tc2(acc1, g1, degp, b1, gamma, beta, W2)
    acc2 = scatter_call(g2, eidx)
    return _tc3(acc2, g2, degp, b2)
